# baseline (speedup 1.0000x reference)
import functools

import jax
import jax.numpy as jnp
from jax import lax
from jax.experimental import pallas as pl
from jax.experimental.pallas import tpu as pltpu
from jax.experimental.pallas import tpu_sc as plsc


def _sc_gather_cols(u2, i2, mfuT, mfiT, mpuT, mpiT, B):
  MF = mfuT.shape[0]
  EMB = mpuT.shape[0]
  info = plsc.get_sparse_core_info()
  NC, NS = info.num_cores, info.num_subcores
  NW = NC * NS
  assert u2.shape == (NW, B // NW)
  bw = B // NW

  mesh = plsc.VectorSubcoreMesh(
      core_axis_name="c", subcore_axis_name="s",
      num_cores=NC, num_subcores=NS)

  @functools.partial(
      pl.kernel,
      out_type=(
          jax.ShapeDtypeStruct((B * MF,), jnp.float32),
          jax.ShapeDtypeStruct((B * MF,), jnp.float32),
          jax.ShapeDtypeStruct((B * EMB,), jnp.float32),
          jax.ShapeDtypeStruct((B * EMB,), jnp.float32),
      ),
      mesh=mesh,
      compiler_params=pltpu.CompilerParams(needs_layout_passes=False),
      scratch_types=(
          [pltpu.VMEM((bw,), jnp.int32)] * 2
          + [pltpu.VMEM((MF, 128), jnp.float32),
             pltpu.VMEM((MF, 128), jnp.float32),
             pltpu.VMEM((EMB, 128), jnp.float32),
             pltpu.VMEM((EMB, 128), jnp.float32)] * 6
          + [pltpu.VMEM((bw * MF,), jnp.float32),
             pltpu.VMEM((bw * MF,), jnp.float32),
             pltpu.VMEM((bw * EMB,), jnp.float32),
             pltpu.VMEM((bw * EMB,), jnp.float32)]
          + [pltpu.SemaphoreType.DMA] * 6
      ),
  )
  def gather_k(u2_hbm, i2_hbm, mfu, mfi, mpu, mpi,
               o_mfu, o_mfi, o_mpu, o_mpi, uix, iix, *rest):
    sets = [tuple(rest[4 * k:4 * k + 4]) for k in range(6)]
    r_mfu, r_mfi, r_mpu, r_mpi = rest[24:28]
    sems = rest[28:34]
    wid = lax.axis_index("s") * NC + lax.axis_index("c")
    base = wid * bw
    pltpu.sync_copy(u2_hbm.at[wid], uix)
    pltpu.sync_copy(i2_hbm.at[wid], iix)
    lane = lax.iota(jnp.int32, 16)
    tabs = (mfu, mfi, mpu, mpi)

    def calc(b):
      g = pl.multiple_of((b >> 4) * 16, 16)
      vu = uix[pl.ds(g, 16)]
      vi = iix[pl.ds(g, 16)]
      m = lane == (b & 15)
      neg = jnp.full((16,), -1, jnp.int32)
      cu = jnp.max(jnp.where(m, vu, neg))
      ci = jnp.max(jnp.where(m, vi, neg))
      return cu, ci

    def fire(cu, ci, bufs, sem):
      cu_t = pl.multiple_of((cu >> 7) * 128, 128)
      ci_t = pl.multiple_of((ci >> 7) * 128, 128)
      offs = (cu_t, ci_t, cu_t, ci_t)
      for tab, off, buf in zip(tabs, offs, bufs):
        pltpu.async_copy(tab.at[:, pl.ds(off, 128)], buf, sem)

    def drain(bufs, sem):
      for tab, buf in zip(tabs, bufs):
        pltpu.make_async_copy(tab.at[:, pl.ds(0, 128)], buf, sem).wait()

    def extract(cu, ci, bufs, b):
      cm_u = jnp.broadcast_to(cu & 127, (16,))
      cm_i = jnp.broadcast_to(ci & 127, (16,))
      row_u = plsc.load_gather(bufs[0], [lane, cm_u])
      row_i = plsc.load_gather(bufs[1], [lane, cm_i])
      o16 = pl.multiple_of(b * MF, MF)
      r_mfu[pl.ds(o16, 16)] = row_u
      r_mfi[pl.ds(o16, 16)] = row_i
      plo_u = plsc.load_gather(bufs[2], [lane, cm_u])
      phi_u = plsc.load_gather(bufs[2], [lane + 16, cm_u])
      plo_i = plsc.load_gather(bufs[3], [lane, cm_i])
      phi_i = plsc.load_gather(bufs[3], [lane + 16, cm_i])
      o32 = pl.multiple_of(b * EMB, EMB)
      r_mpu[pl.ds(o32, 16)] = plo_u
      r_mpu[pl.ds(o32 + 16, 16)] = phi_u
      r_mpi[pl.ds(o32, 16)] = plo_i
      r_mpi[pl.ds(o32 + 16, 16)] = phi_i

    D = 6
    carry0 = []
    for b0 in range(D - 1):
      cu_p, ci_p = calc(b0)
      fire(cu_p, ci_p, sets[b0], sems[b0])
      carry0.extend((cu_p, ci_p))

    def body(b, carry):
      cu_b, ci_b = carry[0], carry[1]
      bn = jnp.minimum(b + (D - 1), bw - 1)
      cu_n, ci_n = calc(bn)
      for k in range(D):
        @pl.when(b % D == k)
        def _(k=k):
          fire(cu_n, ci_n, sets[(k + D - 1) % D], sems[(k + D - 1) % D])
          drain(sets[k], sems[k])
          extract(cu_b, ci_b, sets[k], b)
      return (*carry[2:], cu_n, ci_n)

    lax.fori_loop(0, bw, body, tuple(carry0))
    for j in range(D - 1):
      k = (bw + j) % D
      drain(sets[k], sems[k])

    pltpu.sync_copy(r_mfu, o_mfu.at[pl.ds(base * MF, bw * MF)])
    pltpu.sync_copy(r_mfi, o_mfi.at[pl.ds(base * MF, bw * MF)])
    pltpu.sync_copy(r_mpu, o_mpu.at[pl.ds(base * EMB, bw * EMB)])
    pltpu.sync_copy(r_mpi, o_mpi.at[pl.ds(base * EMB, bw * EMB)])

  return gather_k(u2, i2, mfuT, mfiT, mpuT, mpiT)


def _tc_gather(ut, it, mfuT, mfiT, mpuT, mpiT, n_tc, S=8):
  MF = mfuT.shape[0]
  EMB = mpuT.shape[0]
  grid = n_tc // S

  def imap_u(s):
    return lambda i, uref, iref: (0, uref[i * S + s] >> 7)

  def imap_i(s):
    return lambda i, uref, iref: (0, iref[i * S + s] >> 7)

  in_specs = []
  for s in range(S):
    in_specs += [
        pl.BlockSpec((MF, 128), imap_u(s)),
        pl.BlockSpec((MF, 128), imap_i(s)),
        pl.BlockSpec((EMB, 128), imap_u(s)),
        pl.BlockSpec((EMB, 128), imap_i(s)),
    ]
  R = 128 // S
  out_specs = [
      pl.BlockSpec((MF, 128), lambda i, uref, iref: (0, i // R)),
      pl.BlockSpec((MF, 128), lambda i, uref, iref: (0, i // R)),
      pl.BlockSpec((EMB, 128), lambda i, uref, iref: (0, i // R)),
      pl.BlockSpec((EMB, 128), lambda i, uref, iref: (0, i // R)),
  ]

  def body(uref, iref, *refs):
    blocks = refs[:4 * S]
    outs = refs[4 * S:4 * S + 4]
    i = pl.program_id(0)
    lanes = [lax.broadcasted_iota(jnp.int32, (MF, 128), 1),
             lax.broadcasted_iota(jnp.int32, (MF, 128), 1),
             lax.broadcasted_iota(jnp.int32, (EMB, 128), 1),
             lax.broadcasted_iota(jnp.int32, (EMB, 128), 1)]
    accs = [o[...] for o in outs]
    for s in range(S):
      cu = uref[i * S + s] & 127
      ci = iref[i * S + s] & 127
      t = (i % R) * S + s
      cols = (cu, ci, cu, ci)
      for k in range(4):
        blk = blocks[4 * s + k]
        placed = pltpu.roll(blk[...], t - cols[k], 1)
        accs[k] = jnp.where(lanes[k] == t, placed, accs[k])
    for k in range(4):
      outs[k][...] = accs[k]

  return pl.pallas_call(
      body,
      grid_spec=pltpu.PrefetchScalarGridSpec(
          num_scalar_prefetch=2,
          grid=(grid,),
          in_specs=in_specs,
          out_specs=out_specs,
      ),
      out_shape=[
          jax.ShapeDtypeStruct((MF, n_tc), jnp.float32),
          jax.ShapeDtypeStruct((MF, n_tc), jnp.float32),
          jax.ShapeDtypeStruct((EMB, n_tc), jnp.float32),
          jax.ShapeDtypeStruct((EMB, n_tc), jnp.float32),
      ],
  )(ut, it, *([mfuT, mfiT, mpuT, mpiT] * S))


def _mlp_body_t(mfu_t, mfi_t, mpu_t, mpi_t, w1u, w1i, b1c, w2, b2c, wpm, wph,
                bpc, out):
  ct0 = (((0,), (0,)), ((), ()))
  h = lax.dot_general(w1u[...], mpu_t[...], ct0,
                      preferred_element_type=jnp.float32)
  h += lax.dot_general(w1i[...], mpi_t[...], ct0,
                       preferred_element_type=jnp.float32)
  h = jnp.maximum(h + b1c[...], 0.0)
  h = lax.dot_general(w2[...], h, ct0, preferred_element_type=jnp.float32)
  h = jnp.maximum(h + b2c[...], 0.0)
  mf = mfu_t[...] * mfi_t[...]
  z = lax.dot_general(wpm[...], mf, ct0, preferred_element_type=jnp.float32)
  z += lax.dot_general(wph[...], h, ct0, preferred_element_type=jnp.float32)
  out[...] = jax.nn.sigmoid(z + bpc[...])


def _mlp_body(mfu_r, mfi_r, mpu_r, mpi_r, w1u, w1i, b1r, w2, b2r, wpm, wph,
              bpr, out):
  mfu = mfu_r[...]
  mfi = mfi_r[...]
  mpu = mpu_r[...]
  mpi = mpi_r[...]
  h = jnp.dot(mpu, w1u[...], preferred_element_type=jnp.float32)
  h += jnp.dot(mpi, w1i[...], preferred_element_type=jnp.float32)
  h = jnp.maximum(h + b1r[...], 0.0)
  h = jnp.dot(h, w2[...], preferred_element_type=jnp.float32)
  h = jnp.maximum(h + b2r[...], 0.0)
  mf = mfu * mfi
  z = jnp.dot(mf, wpm[...], preferred_element_type=jnp.float32)
  z += jnp.dot(h, wph[...], preferred_element_type=jnp.float32)
  out[...] = jax.nn.sigmoid(z + bpr[...])


def kernel(user_indices, item_indices, mf_user_table, mf_item_table,
           mlp_user_table, mlp_item_table, W1, b1, W2, b2, Wp, bp):
  B = user_indices.shape[0]
  MF = mf_user_table.shape[1]
  EMB = mlp_user_table.shape[1]
  L1 = W1.shape[1]
  L2 = W2.shape[1]
  NW = 32
  N_TC = 6144
  n_sc = B - N_TC

  ui = user_indices.astype(jnp.int32)
  ii = item_indices.astype(jnp.int32)
  u2 = ui[:n_sc].reshape(NW, n_sc // NW)
  i2 = ii[:n_sc].reshape(NW, n_sc // NW)

  mfuT = mf_user_table.T
  mfiT = mf_item_table.T
  mpuT = mlp_user_table.T
  mpiT = mlp_item_table.T

  f_mfu, f_mfi, f_mpu, f_mpi = _sc_gather_cols(
      u2, i2, mfuT, mfiT, mpuT, mpiT, n_sc)
  t_mfu, t_mfi, t_mpu, t_mpi = _tc_gather(
      ui[n_sc:], ii[n_sc:], mfuT, mfiT, mpuT, mpiT, N_TC)

  mfu = f_mfu.reshape(n_sc, MF)
  mfi = f_mfi.reshape(n_sc, MF)
  mpu = f_mpu.reshape(n_sc, EMB)
  mpi = f_mpi.reshape(n_sc, EMB)

  W1u, W1i = W1[:EMB], W1[EMB:]
  Wp_mf, Wp_h = Wp[:MF], Wp[MF:]
  b1r = b1.reshape(1, L1)
  b2r = b2.reshape(1, L2)
  bpr = bp.reshape(1, 1)
  b1c = b1.reshape(L1, 1)
  b2c = b2.reshape(L2, 1)
  bpc = bp.reshape(1, 1)

  BL = 2048
  full = lambda i: (0, 0)
  rows = lambda i: (i, 0)
  cols = lambda i: (0, i)
  pred1 = pl.pallas_call(
      _mlp_body,
      grid=(n_sc // BL,),
      in_specs=[
          pl.BlockSpec((BL, MF), rows),
          pl.BlockSpec((BL, MF), rows),
          pl.BlockSpec((BL, EMB), rows),
          pl.BlockSpec((BL, EMB), rows),
          pl.BlockSpec((EMB, L1), full),
          pl.BlockSpec((EMB, L1), full),
          pl.BlockSpec((1, L1), full),
          pl.BlockSpec((L1, L2), full),
          pl.BlockSpec((1, L2), full),
          pl.BlockSpec((MF, 1), full),
          pl.BlockSpec((L2, 1), full),
          pl.BlockSpec((1, 1), full),
      ],
      out_specs=pl.BlockSpec((BL, 1), rows),
      out_shape=jax.ShapeDtypeStruct((n_sc, 1), jnp.float32),
  )(mfu, mfi, mpu, mpi, W1u, W1i, b1r, W2, b2r, Wp_mf, Wp_h, bpr)

  pred2 = pl.pallas_call(
      _mlp_body_t,
      grid=(N_TC // BL,),
      in_specs=[
          pl.BlockSpec((MF, BL), cols),
          pl.BlockSpec((MF, BL), cols),
          pl.BlockSpec((EMB, BL), cols),
          pl.BlockSpec((EMB, BL), cols),
          pl.BlockSpec((EMB, L1), full),
          pl.BlockSpec((EMB, L1), full),
          pl.BlockSpec((L1, 1), full),
          pl.BlockSpec((L1, L2), full),
          pl.BlockSpec((L2, 1), full),
          pl.BlockSpec((MF, 1), full),
          pl.BlockSpec((L2, 1), full),
          pl.BlockSpec((1, 1), full),
      ],
      out_specs=pl.BlockSpec((1, BL), cols),
      out_shape=jax.ShapeDtypeStruct((1, N_TC), jnp.float32),
  )(t_mfu, t_mfi, t_mpu, t_mpi, W1u, W1i, b1c, W2, b2c, Wp_mf, Wp_h, bpc)

  return jnp.concatenate([pred1[:, 0], pred2[0]])


# speedup vs baseline: 2.1076x; 2.1076x over previous
import functools

import jax
import jax.numpy as jnp
from jax import lax
from jax.experimental import pallas as pl
from jax.experimental.pallas import tpu as pltpu
from jax.experimental.pallas import tpu_sc as plsc


def _sc_gather_cols(u2, i2, mfuT, mfiT, mpuT, mpiT, B):
  MF = mfuT.shape[0]
  EMB = mpuT.shape[0]
  info = plsc.get_sparse_core_info()
  NC, NS = info.num_cores, info.num_subcores
  NW = NC * NS
  assert u2.shape == (NW, B // NW)
  bw = B // NW

  mesh = plsc.VectorSubcoreMesh(
      core_axis_name="c", subcore_axis_name="s",
      num_cores=NC, num_subcores=NS)

  @functools.partial(
      pl.kernel,
      out_type=(
          jax.ShapeDtypeStruct((B * MF,), jnp.float32),
          jax.ShapeDtypeStruct((B * MF,), jnp.float32),
          jax.ShapeDtypeStruct((B * EMB,), jnp.float32),
          jax.ShapeDtypeStruct((B * EMB,), jnp.float32),
      ),
      mesh=mesh,
      compiler_params=pltpu.CompilerParams(needs_layout_passes=False),
      scratch_types=(
          [pltpu.VMEM((bw,), jnp.int32)] * 2
          + [pltpu.VMEM((MF, 128), jnp.float32),
             pltpu.VMEM((MF, 128), jnp.float32),
             pltpu.VMEM((EMB, 128), jnp.float32),
             pltpu.VMEM((EMB, 128), jnp.float32)] * 6
          + [pltpu.VMEM((bw * MF,), jnp.float32),
             pltpu.VMEM((bw * MF,), jnp.float32),
             pltpu.VMEM((bw * EMB,), jnp.float32),
             pltpu.VMEM((bw * EMB,), jnp.float32)]
          + [pltpu.SemaphoreType.DMA] * 6
      ),
  )
  def gather_k(u2_hbm, i2_hbm, mfu, mfi, mpu, mpi,
               o_mfu, o_mfi, o_mpu, o_mpi, uix, iix, *rest):
    sets = [tuple(rest[4 * k:4 * k + 4]) for k in range(6)]
    r_mfu, r_mfi, r_mpu, r_mpi = rest[24:28]
    sems = rest[28:34]
    wid = lax.axis_index("s") * NC + lax.axis_index("c")
    base = wid * bw
    pltpu.sync_copy(u2_hbm.at[wid], uix)
    pltpu.sync_copy(i2_hbm.at[wid], iix)
    lane = lax.iota(jnp.int32, 16)
    tabs = (mfu, mfi, mpu, mpi)

    def calc(b):
      g = pl.multiple_of((b >> 4) * 16, 16)
      vu = uix[pl.ds(g, 16)]
      vi = iix[pl.ds(g, 16)]
      m = lane == (b & 15)
      neg = jnp.full((16,), -1, jnp.int32)
      cu = jnp.max(jnp.where(m, vu, neg))
      ci = jnp.max(jnp.where(m, vi, neg))
      return cu, ci

    def fire(cu, ci, bufs, sem):
      cu_t = pl.multiple_of((cu >> 7) * 128, 128)
      ci_t = pl.multiple_of((ci >> 7) * 128, 128)
      offs = (cu_t, ci_t, cu_t, ci_t)
      for tab, off, buf in zip(tabs, offs, bufs):
        pltpu.async_copy(tab.at[:, pl.ds(off, 128)], buf, sem)

    def drain(bufs, sem):
      for tab, buf in zip(tabs, bufs):
        pltpu.make_async_copy(tab.at[:, pl.ds(0, 128)], buf, sem).wait()

    def extract(cu, ci, bufs, b):
      cm_u = jnp.broadcast_to(cu & 127, (16,))
      cm_i = jnp.broadcast_to(ci & 127, (16,))
      row_u = plsc.load_gather(bufs[0], [lane, cm_u])
      row_i = plsc.load_gather(bufs[1], [lane, cm_i])
      o16 = pl.multiple_of(b * MF, MF)
      r_mfu[pl.ds(o16, 16)] = row_u
      r_mfi[pl.ds(o16, 16)] = row_i
      plo_u = plsc.load_gather(bufs[2], [lane, cm_u])
      phi_u = plsc.load_gather(bufs[2], [lane + 16, cm_u])
      plo_i = plsc.load_gather(bufs[3], [lane, cm_i])
      phi_i = plsc.load_gather(bufs[3], [lane + 16, cm_i])
      o32 = pl.multiple_of(b * EMB, EMB)
      r_mpu[pl.ds(o32, 16)] = plo_u
      r_mpu[pl.ds(o32 + 16, 16)] = phi_u
      r_mpi[pl.ds(o32, 16)] = plo_i
      r_mpi[pl.ds(o32 + 16, 16)] = phi_i

    D = 6
    carry0 = []
    for b0 in range(D - 1):
      cu_p, ci_p = calc(b0)
      fire(cu_p, ci_p, sets[b0], sems[b0])
      carry0.extend((cu_p, ci_p))

    def body(b, carry):
      cu_b, ci_b = carry[0], carry[1]
      bn = jnp.minimum(b + (D - 1), bw - 1)
      cu_n, ci_n = calc(bn)
      for k in range(D):
        @pl.when(b % D == k)
        def _(k=k):
          fire(cu_n, ci_n, sets[(k + D - 1) % D], sems[(k + D - 1) % D])
          drain(sets[k], sems[k])
          extract(cu_b, ci_b, sets[k], b)
      return (*carry[2:], cu_n, ci_n)

    lax.fori_loop(0, bw, body, tuple(carry0))
    for j in range(D - 1):
      k = (bw + j) % D
      drain(sets[k], sems[k])

    pltpu.sync_copy(r_mfu, o_mfu.at[pl.ds(base * MF, bw * MF)])
    pltpu.sync_copy(r_mfi, o_mfi.at[pl.ds(base * MF, bw * MF)])
    pltpu.sync_copy(r_mpu, o_mpu.at[pl.ds(base * EMB, bw * EMB)])
    pltpu.sync_copy(r_mpi, o_mpi.at[pl.ds(base * EMB, bw * EMB)])

  return gather_k(u2, i2, mfuT, mfiT, mpuT, mpiT)


def _mlp_body(mfu_r, mfi_r, mpu_r, mpi_r, w1u, w1i, b1r, w2, b2r, wpm, wph,
              bpr, out):
  mfu = mfu_r[...]
  mfi = mfi_r[...]
  mpu = mpu_r[...]
  mpi = mpi_r[...]
  h = jnp.dot(mpu, w1u[...], preferred_element_type=jnp.float32)
  h += jnp.dot(mpi, w1i[...], preferred_element_type=jnp.float32)
  h = jnp.maximum(h + b1r[...], 0.0)
  h = jnp.dot(h, w2[...], preferred_element_type=jnp.float32)
  h = jnp.maximum(h + b2r[...], 0.0)
  mf = mfu * mfi
  z = jnp.dot(mf, wpm[...], preferred_element_type=jnp.float32)
  z += jnp.dot(h, wph[...], preferred_element_type=jnp.float32)
  out[...] = jax.nn.sigmoid(z + bpr[...])


def kernel(user_indices, item_indices, mf_user_table, mf_item_table,
           mlp_user_table, mlp_item_table, W1, b1, W2, b2, Wp, bp):
  B = user_indices.shape[0]
  MF = mf_user_table.shape[1]
  EMB = mlp_user_table.shape[1]
  L1 = W1.shape[1]
  L2 = W2.shape[1]
  NW = 32
  n_sc = B

  ui = user_indices.astype(jnp.int32)
  ii = item_indices.astype(jnp.int32)
  u2 = ui.reshape(NW, n_sc // NW)
  i2 = ii.reshape(NW, n_sc // NW)

  f_mfu, f_mfi, f_mpu, f_mpi = _sc_gather_cols(
      u2, i2, mf_user_table.T, mf_item_table.T,
      mlp_user_table.T, mlp_item_table.T, n_sc)

  mfu = f_mfu.reshape(n_sc, MF)
  mfi = f_mfi.reshape(n_sc, MF)
  mpu = f_mpu.reshape(n_sc, EMB)
  mpi = f_mpi.reshape(n_sc, EMB)

  W1u, W1i = W1[:EMB], W1[EMB:]
  Wp_mf, Wp_h = Wp[:MF], Wp[MF:]
  b1r = b1.reshape(1, L1)
  b2r = b2.reshape(1, L2)
  bpr = bp.reshape(1, 1)

  BL = 2048
  full = lambda i: (0, 0)
  rows = lambda i: (i, 0)
  pred1 = pl.pallas_call(
      _mlp_body,
      grid=(n_sc // BL,),
      in_specs=[
          pl.BlockSpec((BL, MF), rows),
          pl.BlockSpec((BL, MF), rows),
          pl.BlockSpec((BL, EMB), rows),
          pl.BlockSpec((BL, EMB), rows),
          pl.BlockSpec((EMB, L1), full),
          pl.BlockSpec((EMB, L1), full),
          pl.BlockSpec((1, L1), full),
          pl.BlockSpec((L1, L2), full),
          pl.BlockSpec((1, L2), full),
          pl.BlockSpec((MF, 1), full),
          pl.BlockSpec((L2, 1), full),
          pl.BlockSpec((1, 1), full),
      ],
      out_specs=pl.BlockSpec((BL, 1), rows),
      out_shape=jax.ShapeDtypeStruct((n_sc, 1), jnp.float32),
  )(mfu, mfi, mpu, mpi, W1u, W1i, b1r, W2, b2r, Wp_mf, Wp_h, bpr)

  return pred1[:, 0]
